# baseline (speedup 1.0000x reference)
import jax
import jax.numpy as jnp
from jax import lax
from jax.experimental import pallas as pl
from jax.experimental.pallas import tpu as pltpu

N0, N1, K = 4096, 20, 1000
B = 64


def _onehot_body(values_ref, idx_ref, out_ref):
    off = values_ref[0]
    on = values_ref[1]
    idx = idx_ref[...]
    kk = lax.broadcasted_iota(jnp.int32, (B, N1, K), 2)
    out_ref[...] = jnp.where(kk == idx[:, :, None], on, off)


def kernel(indices, values):
    return pl.pallas_call(
        _onehot_body,
        grid=(N0 // B,),
        in_specs=[
            pl.BlockSpec(memory_space=pltpu.SMEM),
            pl.BlockSpec((B, N1), lambda i: (i, 0)),
        ],
        out_specs=pl.BlockSpec((B, N1, K), lambda i: (i, 0, 0)),
        out_shape=jax.ShapeDtypeStruct((N0, N1, K), jnp.float32),
    )(values, indices)
